# baseline (speedup 1.0000x reference)
import jax
import jax.numpy as jnp
from jax.experimental import pallas as pl
from jax.experimental.pallas import tpu as pltpu

_NB = 16
_NA = 3
_NC = 80
_NH = 76
_NW = 76
_HW = _NH * _NW
_NBA = _NB * _NA
_AW = (1.25, 2.0, 4.125)
_AH = (1.625, 3.75, 2.875)


def _yolo_block(conf_ref, x_ref, o_ref):
    t = conf_ref[0]

    gx = jax.lax.broadcasted_iota(jnp.int32, (_NH, _NW), 1).astype(jnp.float32)
    gy = jax.lax.broadcasted_iota(jnp.int32, (_NH, _NW), 0).astype(jnp.float32)

    inv_w = 1.0 / _NW
    inv_h = 1.0 / _NH
    for a in range(_NA):
        c0 = a * (5 + _NC)
        aw = _AW[a]
        ah = _AH[a]
        xs = (jax.nn.sigmoid(x_ref[0, c0 + 0]) + gx) * inv_w
        ys = (jax.nn.sigmoid(x_ref[0, c0 + 1]) + gy) * inv_h
        ws = jnp.exp(x_ref[0, c0 + 2]) * (aw * inv_w)
        hs = jnp.exp(x_ref[0, c0 + 3]) * (ah * inv_h)
        det = jax.nn.sigmoid(x_ref[0, c0 + 4])

        m = x_ref[0, c0 + 5]
        idx = jnp.zeros((_NH, _NW), dtype=jnp.float32)
        for i in range(1, _NC):
            c = x_ref[0, c0 + 5 + i]
            gt = c > m
            m = jnp.maximum(m, c)
            idx = jnp.where(gt, jnp.float32(i), idx)
        s = jnp.exp(x_ref[0, c0 + 5] - m)
        for i in range(1, _NC):
            s = s + jnp.exp(x_ref[0, c0 + 5 + i] - m)
        cmax = 1.0 / s

        mask = (det > t).astype(jnp.float32)
        out = jnp.stack([xs, ys, ws, hs, det, cmax, idx], axis=0)
        o_ref[a] = out * mask


_NSPLIT = 4


def _half(output, conf_thresh, half):
    nb = _NB // _NSPLIT
    out = pl.pallas_call(
        _yolo_block,
        grid=(nb,),
        in_specs=[
            pl.BlockSpec(memory_space=pltpu.SMEM),
            pl.BlockSpec((1, _NA * (5 + _NC), _NH, _NW),
                         lambda b: (b + half * nb, 0, 0, 0)),
        ],
        out_specs=pl.BlockSpec((_NA, 7, _NH, _NW), lambda b: (b, 0, 0, 0)),
        out_shape=jax.ShapeDtypeStruct((nb * _NA, 7, _NH, _NW), jnp.float32),
    )(conf_thresh, output)
    return out.reshape(nb * _NA, 7, _HW).transpose(0, 2, 1).reshape(-1, 7)


def kernel(output, conf_thresh):
    parts = [_half(output, conf_thresh, i) for i in range(_NSPLIT)]
    return jnp.concatenate(parts, axis=0)


# speedup vs baseline: 1.0155x; 1.0155x over previous
import jax
import jax.numpy as jnp
from jax.experimental import pallas as pl
from jax.experimental.pallas import tpu as pltpu

_NB = 16
_NA = 3
_NC = 80
_NH = 76
_NW = 76
_HW = _NH * _NW
_NBA = _NB * _NA
_AW = (1.25, 2.0, 4.125)
_AH = (1.625, 3.75, 2.875)


def _yolo_block(conf_ref, x_ref, o_ref):
    t = conf_ref[0]

    gx = jax.lax.broadcasted_iota(jnp.int32, (_NH, _NW), 1).astype(jnp.float32)
    gy = jax.lax.broadcasted_iota(jnp.int32, (_NH, _NW), 0).astype(jnp.float32)

    inv_w = 1.0 / _NW
    inv_h = 1.0 / _NH
    for a in range(_NA):
        c0 = a * (5 + _NC)
        aw = _AW[a]
        ah = _AH[a]
        xs = (jax.nn.sigmoid(x_ref[0, c0 + 0]) + gx) * inv_w
        ys = (jax.nn.sigmoid(x_ref[0, c0 + 1]) + gy) * inv_h
        ws = jnp.exp(x_ref[0, c0 + 2]) * (aw * inv_w)
        hs = jnp.exp(x_ref[0, c0 + 3]) * (ah * inv_h)
        det = jax.nn.sigmoid(x_ref[0, c0 + 4])

        m = x_ref[0, c0 + 5]
        idx = jnp.zeros((_NH, _NW), dtype=jnp.float32)
        for i in range(1, _NC):
            c = x_ref[0, c0 + 5 + i]
            gt = c > m
            m = jnp.maximum(m, c)
            idx = jnp.where(gt, jnp.float32(i), idx)
        s = jnp.exp(x_ref[0, c0 + 5] - m)
        for i in range(1, _NC):
            s = s + jnp.exp(x_ref[0, c0 + 5 + i] - m)
        cmax = 1.0 / s

        mask = (det > t).astype(jnp.float32)
        out = jnp.stack([xs, ys, ws, hs, det, cmax, idx], axis=0)
        o_ref[a] = out * mask


def _span(output, conf_thresh, start, nb):
    out = pl.pallas_call(
        _yolo_block,
        grid=(nb,),
        in_specs=[
            pl.BlockSpec(memory_space=pltpu.SMEM),
            pl.BlockSpec((1, _NA * (5 + _NC), _NH, _NW),
                         lambda b: (b + start, 0, 0, 0)),
        ],
        out_specs=pl.BlockSpec((_NA, 7, _NH, _NW), lambda b: (b, 0, 0, 0)),
        out_shape=jax.ShapeDtypeStruct((nb * _NA, 7, _NH, _NW), jnp.float32),
    )(conf_thresh, output)
    return out.reshape(nb * _NA, 7, _HW).transpose(0, 2, 1).reshape(-1, 7)


def kernel(output, conf_thresh):
    lo = _span(output, conf_thresh, 0, 12)
    hi = _span(output, conf_thresh, 12, 4)
    return jnp.concatenate([lo, hi], axis=0)


# speedup vs baseline: 1.0919x; 1.0753x over previous
import jax
import jax.numpy as jnp
from jax.experimental import pallas as pl
from jax.experimental.pallas import tpu as pltpu

_NB = 16
_NA = 3
_NC = 80
_NH = 76
_NW = 76
_HW = _NH * _NW
_NBA = _NB * _NA
_AW = (1.25, 2.0, 4.125)
_AH = (1.625, 3.75, 2.875)


def _yolo_block(conf_ref, x_ref, o_ref):
    t = conf_ref[0]

    gx = jax.lax.broadcasted_iota(jnp.int32, (_NH, _NW), 1).astype(jnp.float32)
    gy = jax.lax.broadcasted_iota(jnp.int32, (_NH, _NW), 0).astype(jnp.float32)

    inv_w = 1.0 / _NW
    inv_h = 1.0 / _NH
    for a in range(_NA):
        c0 = a * (5 + _NC)
        aw = _AW[a]
        ah = _AH[a]
        xs = (jax.nn.sigmoid(x_ref[0, c0 + 0]) + gx) * inv_w
        ys = (jax.nn.sigmoid(x_ref[0, c0 + 1]) + gy) * inv_h
        ws = jnp.exp(x_ref[0, c0 + 2]) * (aw * inv_w)
        hs = jnp.exp(x_ref[0, c0 + 3]) * (ah * inv_h)
        det = jax.nn.sigmoid(x_ref[0, c0 + 4])

        m = x_ref[0, c0 + 5]
        idx = jnp.zeros((_NH, _NW), dtype=jnp.float32)
        for i in range(1, _NC):
            c = x_ref[0, c0 + 5 + i]
            gt = c > m
            m = jnp.maximum(m, c)
            idx = jnp.where(gt, jnp.float32(i), idx)
        s = jnp.exp(x_ref[0, c0 + 5] - m)
        for i in range(1, _NC):
            s = s + jnp.exp(x_ref[0, c0 + 5 + i] - m)
        cmax = 1.0 / s

        mask = (det > t).astype(jnp.float32)
        out = jnp.stack([xs, ys, ws, hs, det, cmax, idx], axis=0)
        o_ref[a] = out * mask


def _span(output, conf_thresh, start, nb):
    out = pl.pallas_call(
        _yolo_block,
        grid=(nb,),
        in_specs=[
            pl.BlockSpec(memory_space=pltpu.SMEM),
            pl.BlockSpec((1, _NA * (5 + _NC), _NH, _NW),
                         lambda b: (b + start, 0, 0, 0)),
        ],
        out_specs=pl.BlockSpec((_NA, 7, _NH, _NW), lambda b: (b, 0, 0, 0)),
        out_shape=jax.ShapeDtypeStruct((nb * _NA, 7, _NH, _NW), jnp.float32),
    )(conf_thresh, output)
    return out.reshape(nb * _NA, 7, _HW).transpose(0, 2, 1).reshape(-1, 7)


def kernel(output, conf_thresh):
    lo = _span(output, conf_thresh, 0, 8)
    hi = _span(output, conf_thresh, 8, 8)
    return jnp.concatenate([lo, hi], axis=0)
